# Initial kernel scaffold; baseline (speedup 1.0000x reference)
import jax
import jax.numpy as jnp
from jax.experimental import pallas as pl


def kernel(x, edge_index, Wp_in, bp_in, Wl_e, Wr_e, att_e, b_e, Wl_d, Wr_d, att_d, b_d, Wp_out, bp_out):
    raise NotImplementedError("write your pallas kernel here")


# speedup vs baseline: 8.9417x; 8.9417x over previous
import functools

import jax
import jax.numpy as jnp
from jax import lax
from jax.experimental import pallas as pl
from jax.experimental.pallas import tpu as pltpu
from jax.experimental.pallas import tpu_sc as plsc

N = 10000
E = 320000
D_IN = 128
D0 = 256
D1 = 128

NC, NS, L = 2, 16, 16
NW = NC * NS
NP = 10240
EP = 327680
EPW = EP // NW
RPT = NP // NS

NEG = 0.2
EPS = 1e-16


def _make_edge_agg(ds, share_agg, C):
    nj = ds // L
    na = 128 // L
    NCHUNK = EPW // C

    def body(tl, tr, ta, srcr, dstr, att_h,
             num_o, den_o,
             *scratch):
        if share_agg:
            (src_v, dst_v, rows_l, rows_r, att_v, g_v,
             zrow, acc_sp, den_sp, sem) = scratch
            rows_a = rows_l
        else:
            (src_v, dst_v, rows_l, rows_r, rows_a, att_v, g_v,
             zrow, acc_sp, den_sp, sem) = scratch
        cid = lax.axis_index("c")
        sid = lax.axis_index("s")
        wid = cid * NS + sid

        zv = jnp.zeros((L,), jnp.float32)

        def zr(r, _):
            for j in range(na):
                rows_a[r, pl.ds(j * L, L)] = zv
            return 0
        lax.fori_loop(0, C, zr, 0)
        def zc(k, _):
            zrow[pl.ds(k * L, L)] = zv
            return 0
        lax.fori_loop(0, RPT // L, zc, 0)

        for k in range(RPT // C):
            pltpu.sync_copy(rows_a, acc_sp.at[pl.ds(sid * RPT + k * C, C)])
        pltpu.sync_copy(zrow, den_sp.at[pl.ds(sid * RPT, RPT)])

        pltpu.sync_copy(att_h, att_v)
        att_c = [att_v[pl.ds(j * L, L)] for j in range(nj)]

        plsc.subcore_barrier()

        def chunk(i, _):
            base = wid * EPW + i * C
            pltpu.sync_copy(srcr.at[pl.ds(base, C)], src_v)
            pltpu.sync_copy(dstr.at[pl.ds(base, C)], dst_v)
            cps = [pltpu.async_copy(tl.at[src_v], rows_l, sem),
                   pltpu.async_copy(tr.at[dst_v], rows_r, sem)]
            if not share_agg:
                cps.append(pltpu.async_copy(ta.at[src_v], rows_a, sem))
            for cp in cps:
                cp.wait()

            agg = rows_l if share_agg else rows_a
            lane = lax.iota(jnp.int32, L)

            def grp(cb, _):
                g_acc = jnp.zeros((L,), jnp.float32)
                for k in range(L):
                    c = cb * L + k
                    acc = None
                    for j in range(nj):
                        a = rows_l[c, pl.ds(j * L, L)]
                        b = rows_r[c, pl.ds(j * L, L)]
                        t = a + b
                        t = jnp.maximum(t, NEG * t)
                        acc = (t * att_c[j] if acc is None
                               else acc + t * att_c[j])
                    gb = jnp.exp(jnp.full((L,), jnp.sum(acc)))
                    g_acc = jnp.where(lane == k, gb, g_acc)
                    for j in range(na):
                        agg[c, pl.ds(j * L, L)] = agg[c, pl.ds(j * L, L)] * gb
                g_v[pl.ds(cb * L, L)] = g_acc
                return 0
            lax.fori_loop(0, C // L, grp, 0)

            pltpu.sync_copy(agg, acc_sp.at[dst_v], add=True)
            pltpu.sync_copy(g_v, den_sp.at[dst_v], add=True)
            return 0
        lax.fori_loop(0, NCHUNK, chunk, 0)

        plsc.subcore_barrier()

        pltpu.sync_copy(acc_sp.at[pl.ds(sid * RPT, RPT)],
                        num_o.at[cid, pl.ds(sid * RPT, RPT)])
        pltpu.sync_copy(den_sp.at[pl.ds(sid * RPT, RPT)],
                        den_o.at[cid, pl.ds(sid * RPT, RPT)])

    mesh = plsc.VectorSubcoreMesh(core_axis_name="c", subcore_axis_name="s",
                                  num_cores=NC, num_subcores=NS)
    scratch = [
        pltpu.VMEM((C,), jnp.int32),
        pltpu.VMEM((C,), jnp.int32),
        pltpu.VMEM((C, ds), jnp.float32),
        pltpu.VMEM((C, ds), jnp.float32),
        pltpu.VMEM((C, 128), jnp.float32),
        pltpu.VMEM((ds,), jnp.float32),
        pltpu.VMEM((C,), jnp.float32),
        pltpu.VMEM((RPT,), jnp.float32),
        pltpu.VMEM_SHARED((NP, 128), jnp.float32),
        pltpu.VMEM_SHARED((NP,), jnp.float32),
        pltpu.SemaphoreType.DMA,
    ]
    if share_agg:
        del scratch[4]
    return pl.kernel(
        body,
        out_type=[jax.ShapeDtypeStruct((NC, NP, 128), jnp.float32),
                  jax.ShapeDtypeStruct((NC, NP), jnp.float32)],
        mesh=mesh,
        compiler_params=pltpu.CompilerParams(needs_layout_passes=False),
        scratch_types=scratch,
        name=f"edge_agg_d{ds}",
    )


def _dgt(a, b):
    return lax.dot_general(a, b, (((1,), (1,)), ((), ())),
                           preferred_element_type=jnp.float32)


def _prep_body(wp_out, wl_d, b_d, bp_out, m_o, cvec_o):
    m_o[...] = jnp.dot(wp_out[...], wl_d[...],
                       preferred_element_type=jnp.float32)
    cvec_o[...] = _dgt(b_d[...], wp_out[...]) + bp_out[...]


def _enc_tables_body(x, wp_in, bp_in, wl_e, wr_e, xl_o, xr_o):
    h = _dgt(x[...], wp_in[...]) + bp_in[...]
    xl_o[...] = _dgt(h, wl_e[...])
    xr_o[...] = _dgt(h, wr_e[...])


def _combine_enc_body(p, den, xl, xr, att, b_e, wl_d, wr_d, m,
                      z_o, xld_o, xrd_o, y_o):
    t = xl[...] + xr[...]
    t = jnp.maximum(t, NEG * t)
    g = jnp.exp(jnp.sum(t * att[...], axis=1, keepdims=True))
    num = p[0] + p[1] + g * xl[...]
    dent = den[0] + den[1] + g + EPS
    z = num / dent + b_e[...]
    z_o[...] = z
    xld_o[...] = _dgt(z, wl_d[...])
    xrd_o[...] = _dgt(z, wr_d[...])
    y_o[...] = _dgt(z, m[...])


def _final_body(p, den, xl, xr, y, att, cvec, xhat_o):
    t = xl[...] + xr[...]
    t = jnp.maximum(t, NEG * t)
    g = jnp.exp(jnp.sum(t * att[...], axis=1, keepdims=True))
    num = p[0] + p[1] + g * y[...]
    dent = den[0] + den[1] + g + EPS
    xhat_o[...] = num / dent + cvec[...]


def _full(shape):
    return pl.BlockSpec(shape, lambda i: (0,) * len(shape))


def _rows(bm, *rest):
    return pl.BlockSpec((bm,) + rest, lambda i: (i,) + (0,) * len(rest))


def _rows3(bm, d):
    return pl.BlockSpec((2, bm, d), lambda i: (0, i, 0))


def kernel(x, edge_index, Wp_in, bp_in, Wl_e, Wr_e, att_e, b_e,
           Wl_d, Wr_d, att_d, b_d, Wp_out, bp_out):
    f32 = jnp.float32
    src = edge_index[0]
    dst = edge_index[1]
    pad_idx = N + (jnp.arange(EP - E, dtype=jnp.int32) % (NP - N))
    srcp = jnp.concatenate([src, pad_idx])
    dstp = jnp.concatenate([dst, pad_idx])
    x_pad = jnp.pad(x, ((0, NP - N), (0, 0)))

    m, cvec = pl.pallas_call(
        _prep_body,
        out_shape=[jax.ShapeDtypeStruct((D_IN, D1), f32),
                   jax.ShapeDtypeStruct((1, D_IN), f32)],
    )(Wp_out, Wl_d, b_d.reshape(1, D0), bp_out.reshape(1, D_IN))

    BM = 512
    grid = (NP // BM,)
    xl_e, xr_e = pl.pallas_call(
        _enc_tables_body,
        grid=grid,
        in_specs=[_rows(BM, D_IN), _full((D0, D_IN)), _full((1, D0)),
                  _full((D1, D0)), _full((D1, D0))],
        out_specs=[_rows(BM, D1), _rows(BM, D1)],
        out_shape=[jax.ShapeDtypeStruct((NP, D1), f32),
                   jax.ShapeDtypeStruct((NP, D1), f32)],
    )(x_pad, Wp_in, bp_in.reshape(1, D0), Wl_e, Wr_e)

    p_e, d_e = _make_edge_agg(D1, True, 128)(xl_e, xr_e, xl_e, srcp, dstp,
                                             att_e)
    d_e = d_e.reshape(NC, NP, 1)

    z_pad, xl_d, xr_d, y = pl.pallas_call(
        _combine_enc_body,
        grid=grid,
        in_specs=[_rows3(BM, D1), _rows3(BM, 1), _rows(BM, D1), _rows(BM, D1),
                  _full((1, D1)), _full((1, D1)),
                  _full((D0, D1)), _full((D0, D1)), _full((D_IN, D1))],
        out_specs=[_rows(BM, D1), _rows(BM, D0), _rows(BM, D0),
                   _rows(BM, D_IN)],
        out_shape=[jax.ShapeDtypeStruct((NP, D1), f32),
                   jax.ShapeDtypeStruct((NP, D0), f32),
                   jax.ShapeDtypeStruct((NP, D0), f32),
                   jax.ShapeDtypeStruct((NP, D_IN), f32)],
    )(p_e, d_e, xl_e, xr_e, att_e.reshape(1, D1), b_e.reshape(1, D1),
      Wl_d, Wr_d, m)

    p_d, d_d = _make_edge_agg(D0, False, 32)(xl_d, xr_d, y, srcp, dstp, att_d)
    d_d = d_d.reshape(NC, NP, 1)

    BF = 400
    x_hat = pl.pallas_call(
        _final_body,
        grid=(N // BF,),
        in_specs=[_rows3(BF, D_IN), _rows3(BF, 1), _rows(BF, D0),
                  _rows(BF, D0), _rows(BF, D_IN),
                  _full((1, D0)), _full((1, D_IN))],
        out_specs=_rows(BF, D_IN),
        out_shape=jax.ShapeDtypeStruct((N, D_IN), f32),
    )(p_d, d_d, xl_d, xr_d, y, att_d.reshape(1, D0), cvec)

    return (x_hat, z_pad[:N])


# speedup vs baseline: 11.8924x; 1.3300x over previous
import functools

import jax
import jax.numpy as jnp
from jax import lax
from jax.experimental import pallas as pl
from jax.experimental.pallas import tpu as pltpu
from jax.experimental.pallas import tpu_sc as plsc

N = 10000
E = 320000
D_IN = 128
D0 = 256
D1 = 128

NC, NS, L = 2, 16, 16
NW = NC * NS
NP = 10240
EP = 327680
EPW = EP // NW
RPT = NP // NS

NEG = 0.2
EPS = 1e-16


def _make_edge_agg(ds, share_agg, C):
    nj = ds // L
    na = 128 // L
    NCHUNK = EPW // C
    assert NCHUNK % 2 == 0

    def body(tl, tr, ta, srcr, dstr, att_h,
             num_o, den_o,
             *scratch):
        if share_agg:
            (s0, s1, d0, d1, l0, l1, r0, r1, att_v, g_v,
             zrow, acc_sp, den_sp, sem_g, sem_i) = scratch
            a0, a1 = l0, l1
        else:
            (s0, s1, d0, d1, l0, l1, r0, r1, a0, a1, att_v, g_v,
             zrow, acc_sp, den_sp, sem_g, sem_i) = scratch
        src = (s0, s1)
        dst = (d0, d1)
        rl = (l0, l1)
        rr = (r0, r1)
        ra = (a0, a1)
        cid = lax.axis_index("c")
        sid = lax.axis_index("s")
        wid = cid * NS + sid
        ebase = wid * EPW

        zv = jnp.zeros((L,), jnp.float32)

        def zr(r, _):
            for j in range(na):
                ra[0][r, pl.ds(j * L, L)] = zv
            return 0
        lax.fori_loop(0, C, zr, 0)
        def zc(k, _):
            zrow[pl.ds(k * L, L)] = zv
            return 0
        lax.fori_loop(0, RPT // L, zc, 0)

        for k in range(RPT // C):
            pltpu.sync_copy(ra[0], acc_sp.at[pl.ds(sid * RPT + k * C, C)])
        pltpu.sync_copy(zrow, den_sp.at[pl.ds(sid * RPT, RPT)])

        pltpu.sync_copy(att_h, att_v)
        att_c = [att_v[pl.ds(j * L, L)] for j in range(nj)]
        lane = lax.iota(jnp.int32, L)

        plsc.subcore_barrier()

        def idx_start(i, b):
            base = ebase + lax.rem(i, NCHUNK) * C
            pltpu.async_copy(srcr.at[pl.ds(base, C)], src[b], sem_i)
            pltpu.async_copy(dstr.at[pl.ds(base, C)], dst[b], sem_i)

        def idx_wait(b):
            pltpu.make_async_copy(srcr.at[pl.ds(0, C)], src[b], sem_i).wait()
            pltpu.make_async_copy(dstr.at[pl.ds(0, C)], dst[b], sem_i).wait()

        def gather_start(b):
            pltpu.async_copy(tl.at[src[b]], rl[b], sem_g)
            pltpu.async_copy(tr.at[dst[b]], rr[b], sem_g)
            if not share_agg:
                pltpu.async_copy(ta.at[src[b]], ra[b], sem_g)

        def gather_wait(b):
            pltpu.make_async_copy(tl.at[src[b]], rl[b], sem_g).wait()
            pltpu.make_async_copy(tr.at[dst[b]], rr[b], sem_g).wait()
            if not share_agg:
                pltpu.make_async_copy(ta.at[src[b]], ra[b], sem_g).wait()

        def compute(b):
            rows_l, rows_r, agg = rl[b], rr[b], ra[b]

            def grp(cb, _):
                g_acc = jnp.zeros((L,), jnp.float32)
                for k in range(L):
                    c = cb * L + k
                    acc = None
                    for j in range(nj):
                        a = rows_l[c, pl.ds(j * L, L)]
                        bb = rows_r[c, pl.ds(j * L, L)]
                        t = a + bb
                        t = jnp.maximum(t, NEG * t)
                        acc = (t * att_c[j] if acc is None
                               else acc + t * att_c[j])
                    gb = jnp.exp(jnp.full((L,), jnp.sum(acc)))
                    g_acc = jnp.where(lane == k, gb, g_acc)
                    for j in range(na):
                        agg[c, pl.ds(j * L, L)] = agg[c, pl.ds(j * L, L)] * gb
                g_v[pl.ds(cb * L, L)] = g_acc
                return 0
            lax.fori_loop(0, C // L, grp, 0)

        idx_start(0, 0)
        idx_wait(0)
        gather_start(0)
        idx_start(1, 1)

        def pair(g2, _):
            for b in (0, 1):
                i = 2 * g2 + b
                nb = (b + 1) % 2
                gather_wait(b)
                idx_wait(nb)
                gather_start(nb)
                compute(b)
                pltpu.sync_copy(ra[b], acc_sp.at[dst[b]], add=True)
                pltpu.sync_copy(g_v, den_sp.at[dst[b]], add=True)
                idx_start(i + 2, b)
            return 0
        lax.fori_loop(0, NCHUNK // 2, pair, 0)

        gather_wait(0)
        idx_wait(1)

        plsc.subcore_barrier()

        pltpu.sync_copy(acc_sp.at[pl.ds(sid * RPT, RPT)],
                        num_o.at[cid, pl.ds(sid * RPT, RPT)])
        pltpu.sync_copy(den_sp.at[pl.ds(sid * RPT, RPT)],
                        den_o.at[cid, pl.ds(sid * RPT, RPT)])

    mesh = plsc.VectorSubcoreMesh(core_axis_name="c", subcore_axis_name="s",
                                  num_cores=NC, num_subcores=NS)
    scratch = [
        pltpu.VMEM((C,), jnp.int32),
        pltpu.VMEM((C,), jnp.int32),
        pltpu.VMEM((C,), jnp.int32),
        pltpu.VMEM((C,), jnp.int32),
        pltpu.VMEM((C, ds), jnp.float32),
        pltpu.VMEM((C, ds), jnp.float32),
        pltpu.VMEM((C, ds), jnp.float32),
        pltpu.VMEM((C, ds), jnp.float32),
        pltpu.VMEM((C, 128), jnp.float32),
        pltpu.VMEM((C, 128), jnp.float32),
        pltpu.VMEM((ds,), jnp.float32),
        pltpu.VMEM((C,), jnp.float32),
        pltpu.VMEM((RPT,), jnp.float32),
        pltpu.VMEM_SHARED((NP, 128), jnp.float32),
        pltpu.VMEM_SHARED((NP,), jnp.float32),
        pltpu.SemaphoreType.DMA,
        pltpu.SemaphoreType.DMA,
    ]
    if share_agg:
        del scratch[8:10]
    return pl.kernel(
        body,
        out_type=[jax.ShapeDtypeStruct((NC, NP, 128), jnp.float32),
                  jax.ShapeDtypeStruct((NC, NP), jnp.float32)],
        mesh=mesh,
        compiler_params=pltpu.CompilerParams(needs_layout_passes=False),
        scratch_types=scratch,
        name=f"edge_agg_d{ds}",
    )


def _dgt(a, b):
    return lax.dot_general(a, b, (((1,), (1,)), ((), ())),
                           preferred_element_type=jnp.float32)


def _prep_body(wp_out, wl_d, b_d, bp_out, m_o, cvec_o):
    m_o[...] = jnp.dot(wp_out[...], wl_d[...],
                       preferred_element_type=jnp.float32)
    cvec_o[...] = _dgt(b_d[...], wp_out[...]) + bp_out[...]


def _enc_tables_body(x, wp_in, bp_in, wl_e, wr_e, xl_o, xr_o):
    h = _dgt(x[...], wp_in[...]) + bp_in[...]
    xl_o[...] = _dgt(h, wl_e[...])
    xr_o[...] = _dgt(h, wr_e[...])


def _combine_enc_body(p, den, xl, xr, att, b_e, wl_d, wr_d, m,
                      z_o, xld_o, xrd_o, y_o):
    t = xl[...] + xr[...]
    t = jnp.maximum(t, NEG * t)
    g = jnp.exp(jnp.sum(t * att[...], axis=1, keepdims=True))
    num = p[0] + p[1] + g * xl[...]
    dent = den[0] + den[1] + g + EPS
    z = num / dent + b_e[...]
    z_o[...] = z
    xld_o[...] = _dgt(z, wl_d[...])
    xrd_o[...] = _dgt(z, wr_d[...])
    y_o[...] = _dgt(z, m[...])


def _final_body(p, den, xl, xr, y, att, cvec, xhat_o):
    t = xl[...] + xr[...]
    t = jnp.maximum(t, NEG * t)
    g = jnp.exp(jnp.sum(t * att[...], axis=1, keepdims=True))
    num = p[0] + p[1] + g * y[...]
    dent = den[0] + den[1] + g + EPS
    xhat_o[...] = num / dent + cvec[...]


def _full(shape):
    return pl.BlockSpec(shape, lambda i: (0,) * len(shape))


def _rows(bm, *rest):
    return pl.BlockSpec((bm,) + rest, lambda i: (i,) + (0,) * len(rest))


def _rows3(bm, d):
    return pl.BlockSpec((2, bm, d), lambda i: (0, i, 0))


def kernel(x, edge_index, Wp_in, bp_in, Wl_e, Wr_e, att_e, b_e,
           Wl_d, Wr_d, att_d, b_d, Wp_out, bp_out):
    f32 = jnp.float32
    src = edge_index[0]
    dst = edge_index[1]
    pad_idx = N + (jnp.arange(EP - E, dtype=jnp.int32) % (NP - N))
    srcp = jnp.concatenate([src, pad_idx])
    dstp = jnp.concatenate([dst, pad_idx])
    x_pad = jnp.pad(x, ((0, NP - N), (0, 0)))

    m, cvec = pl.pallas_call(
        _prep_body,
        out_shape=[jax.ShapeDtypeStruct((D_IN, D1), f32),
                   jax.ShapeDtypeStruct((1, D_IN), f32)],
    )(Wp_out, Wl_d, b_d.reshape(1, D0), bp_out.reshape(1, D_IN))

    BM = 512
    grid = (NP // BM,)
    xl_e, xr_e = pl.pallas_call(
        _enc_tables_body,
        grid=grid,
        in_specs=[_rows(BM, D_IN), _full((D0, D_IN)), _full((1, D0)),
                  _full((D1, D0)), _full((D1, D0))],
        out_specs=[_rows(BM, D1), _rows(BM, D1)],
        out_shape=[jax.ShapeDtypeStruct((NP, D1), f32),
                   jax.ShapeDtypeStruct((NP, D1), f32)],
    )(x_pad, Wp_in, bp_in.reshape(1, D0), Wl_e, Wr_e)

    p_e, d_e = _make_edge_agg(D1, True, 80)(xl_e, xr_e, xl_e, srcp, dstp,
                                            att_e)
    d_e = d_e.reshape(NC, NP, 1)

    z_pad, xl_d, xr_d, y = pl.pallas_call(
        _combine_enc_body,
        grid=grid,
        in_specs=[_rows3(BM, D1), _rows3(BM, 1), _rows(BM, D1), _rows(BM, D1),
                  _full((1, D1)), _full((1, D1)),
                  _full((D0, D1)), _full((D0, D1)), _full((D_IN, D1))],
        out_specs=[_rows(BM, D1), _rows(BM, D0), _rows(BM, D0),
                   _rows(BM, D_IN)],
        out_shape=[jax.ShapeDtypeStruct((NP, D1), f32),
                   jax.ShapeDtypeStruct((NP, D0), f32),
                   jax.ShapeDtypeStruct((NP, D0), f32),
                   jax.ShapeDtypeStruct((NP, D_IN), f32)],
    )(p_e, d_e, xl_e, xr_e, att_e.reshape(1, D1), b_e.reshape(1, D1),
      Wl_d, Wr_d, m)

    p_d, d_d = _make_edge_agg(D0, False, 32)(xl_d, xr_d, y, srcp, dstp, att_d)
    d_d = d_d.reshape(NC, NP, 1)

    BF = 400
    x_hat = pl.pallas_call(
        _final_body,
        grid=(N // BF,),
        in_specs=[_rows3(BF, D_IN), _rows3(BF, 1), _rows(BF, D0),
                  _rows(BF, D0), _rows(BF, D_IN),
                  _full((1, D0)), _full((1, D_IN))],
        out_specs=_rows(BF, D_IN),
        out_shape=jax.ShapeDtypeStruct((N, D_IN), f32),
    )(p_d, d_d, xl_d, xr_d, y, att_d.reshape(1, D0), cvec)

    return (x_hat, z_pad[:N])


# speedup vs baseline: 16.2786x; 1.3688x over previous
import functools

import jax
import jax.numpy as jnp
from jax import lax
from jax.experimental import pallas as pl
from jax.experimental.pallas import tpu as pltpu
from jax.experimental.pallas import tpu_sc as plsc

N = 10000
E = 320000
D_IN = 128
D0 = 256
D1 = 128

NC, NS, L = 2, 16, 16
NW = NC * NS
NP = 10240
EP = 327680
EPW = EP // NW
RPT = NP // NS

NEG = 0.2
EPS = 1e-16


def _make_edge_agg(ds, C, packed):
    nj2 = ds // (2 * L)
    nj = ds // L
    na = 128 // L
    NCHUNK = EPW // C
    assert NCHUNK % 2 == 0 and C % L == 0

    def body(tlb, trb, ta, ep, att_h,
             num_o, den_o,
             *scratch):
        if packed:
            (p0, p1, l0, l1, r0, r1, a0, a1, att_v, g_v,
             acc_sp, den_sp, sem_g, sem_i) = scratch
        else:
            (p0, p1, l0, l1, r0, r1, att_v, g_v,
             acc_sp, den_sp, sem_g, sem_i) = scratch
            a0, a1 = l0, l1
        sd = (p0, p1)
        rl = (l0, l1)
        rr = (r0, r1)
        ra = (a0, a1)
        cid = lax.axis_index("c")
        sid = lax.axis_index("s")
        wid = cid * NS + sid
        rbase = wid * NCHUNK

        zv = jnp.zeros((L,), jnp.float32)

        def zr(r, _):
            for j in range(na):
                ra[0][r, pl.ds(j * L, L)] = zv
            return 0
        lax.fori_loop(0, C, zr, 0)
        def zc(k, _):
            g_v[pl.ds(k * L, L)] = zv
            return 0
        lax.fori_loop(0, C // L, zc, 0)

        for k in range(RPT // C):
            pltpu.sync_copy(ra[0], acc_sp.at[pl.ds(sid * RPT + k * C, C)])
            pltpu.sync_copy(g_v, den_sp.at[pl.ds(sid * RPT + k * C, C)])

        pltpu.sync_copy(att_h, att_v)
        if packed:
            att_c = [plsc.bitcast(att_v[pl.ds(j * L, L)], jnp.bfloat16)
                     for j in range(nj2)]
        else:
            att_c = [att_v[pl.ds(j * L, L)] for j in range(nj)]
        lane = lax.iota(jnp.int32, L)

        plsc.subcore_barrier()

        def idx_start(i, b):
            row = rbase + lax.rem(i, NCHUNK)
            pltpu.async_copy(ep.at[row], sd[b], sem_i)

        def idx_wait(b):
            pltpu.make_async_copy(ep.at[0], sd[b], sem_i).wait()

        def gather_start(b):
            pltpu.async_copy(tlb.at[sd[b].at[0]], rl[b], sem_g)
            pltpu.async_copy(trb.at[sd[b].at[1]], rr[b], sem_g)
            if packed:
                pltpu.async_copy(ta.at[sd[b].at[0]], ra[b], sem_g)

        def gather_wait(b):
            pltpu.make_async_copy(tlb.at[sd[b].at[0]], rl[b], sem_g).wait()
            pltpu.make_async_copy(trb.at[sd[b].at[1]], rr[b], sem_g).wait()
            if packed:
                pltpu.make_async_copy(ta.at[sd[b].at[0]], ra[b], sem_g).wait()

        def compute(b):
            rows_l, rows_r, agg = rl[b], rr[b], ra[b]

            def grp(cb, _):
                g_acc = jnp.zeros((L,), jnp.float32)
                for k in range(L):
                    c = cb * L + k
                    acc = None
                    if packed:
                        for j in range(nj2):
                            a = plsc.bitcast(rows_l[c, pl.ds(j * L, L)],
                                             jnp.bfloat16)
                            bb = plsc.bitcast(rows_r[c, pl.ds(j * L, L)],
                                              jnp.bfloat16)
                            t = a + bb
                            t = jnp.maximum(t, NEG * t)
                            q0, q1 = plsc.unpack(
                                t * att_c[j],
                                format=plsc.PackFormat.INTERLEAVED)
                            acc = q0 + q1 if acc is None else acc + q0 + q1
                    else:
                        for j in range(nj):
                            a = rows_l[c, pl.ds(j * L, L)]
                            bb = rows_r[c, pl.ds(j * L, L)]
                            t = a + bb
                            t = jnp.maximum(t, NEG * t)
                            acc = (t * att_c[j] if acc is None
                                   else acc + t * att_c[j])
                    gb = jnp.exp(jnp.full((L,), jnp.sum(acc)))
                    g_acc = jnp.where(lane == k, gb, g_acc)
                    for j in range(na):
                        agg[c, pl.ds(j * L, L)] = agg[c, pl.ds(j * L, L)] * gb
                g_v[pl.ds(cb * L, L)] = g_acc
                return 0
            lax.fori_loop(0, C // L, grp, 0)

        idx_start(0, 0)
        idx_wait(0)
        gather_start(0)
        idx_start(1, 1)

        def pair(g2, _):
            for b in (0, 1):
                i = 2 * g2 + b
                nb = (b + 1) % 2
                gather_wait(b)
                idx_wait(nb)
                gather_start(nb)
                compute(b)
                pltpu.sync_copy(ra[b], acc_sp.at[sd[b].at[1]], add=True)
                pltpu.sync_copy(g_v, den_sp.at[sd[b].at[1]], add=True)
                idx_start(i + 2, b)
            return 0
        lax.fori_loop(0, NCHUNK // 2, pair, 0)

        gather_wait(0)
        idx_wait(1)

        plsc.subcore_barrier()

        pltpu.sync_copy(acc_sp.at[pl.ds(sid * RPT, RPT)],
                        num_o.at[cid, pl.ds(sid * RPT, RPT)])
        pltpu.sync_copy(den_sp.at[pl.ds(sid * RPT, RPT)],
                        den_o.at[cid, pl.ds(sid * RPT, RPT)])

    mesh = plsc.VectorSubcoreMesh(core_axis_name="c", subcore_axis_name="s",
                                  num_cores=NC, num_subcores=NS)
    dsw = ds // 2 if packed else ds
    scratch = [
        pltpu.VMEM((2, C), jnp.int32),
        pltpu.VMEM((2, C), jnp.int32),
        pltpu.VMEM((C, dsw), jnp.float32),
        pltpu.VMEM((C, dsw), jnp.float32),
        pltpu.VMEM((C, dsw), jnp.float32),
        pltpu.VMEM((C, dsw), jnp.float32),
        pltpu.VMEM((C, 128), jnp.float32),
        pltpu.VMEM((C, 128), jnp.float32),
        pltpu.VMEM((dsw,), jnp.float32),
        pltpu.VMEM((C,), jnp.float32),
        pltpu.VMEM_SHARED((NP, 128), jnp.float32),
        pltpu.VMEM_SHARED((NP,), jnp.float32),
        pltpu.SemaphoreType.DMA,
        pltpu.SemaphoreType.DMA,
    ]
    if not packed:
        del scratch[6:8]
    return pl.kernel(
        body,
        out_type=[jax.ShapeDtypeStruct((NC, NP, 128), jnp.float32),
                  jax.ShapeDtypeStruct((NC, NP), jnp.float32)],
        mesh=mesh,
        compiler_params=pltpu.CompilerParams(needs_layout_passes=False),
        scratch_types=scratch,
        name=f"edge_agg_d{ds}",
    )


def _dgt(a, b):
    return lax.dot_general(a, b, (((1,), (1,)), ((), ())),
                           preferred_element_type=jnp.float32)


def _prep_body(wp_out, wl_d, b_d, bp_out, m_o, cvec_o):
    m_o[...] = jnp.dot(wp_out[...], wl_d[...],
                       preferred_element_type=jnp.float32)
    cvec_o[...] = _dgt(b_d[...], wp_out[...]) + bp_out[...]


def _enc_tables_body(x, wp_in, bp_in, wl_e, wr_e, xl_o, xr_o):
    h = _dgt(x[...], wp_in[...]) + bp_in[...]
    xl_o[...] = _dgt(h, wl_e[...])
    xr_o[...] = _dgt(h, wr_e[...])


def _combine_enc_body(p, den, xl, xr, att, b_e, wl_d, wr_d, m,
                      z_o, xld_o, xrd_o, y_o):
    t = xl[...] + xr[...]
    t = jnp.maximum(t, NEG * t)
    g = jnp.exp(jnp.sum(t * att[...], axis=1, keepdims=True))
    num = p[0] + p[1] + g * xl[...]
    dent = den[0] + den[1] + g + EPS
    z = num / dent + b_e[...]
    z_o[...] = z
    xld_o[...] = _dgt(z, wl_d[...])
    xrd_o[...] = _dgt(z, wr_d[...])
    y_o[...] = _dgt(z, m[...])


def _final_body(p, den, xl, xr, y, att, cvec, xhat_o):
    t = xl[...] + xr[...]
    t = jnp.maximum(t, NEG * t)
    g = jnp.exp(jnp.sum(t * att[...], axis=1, keepdims=True))
    num = p[0] + p[1] + g * y[...]
    dent = den[0] + den[1] + g + EPS
    xhat_o[...] = num / dent + cvec[...]


def _full(shape):
    return pl.BlockSpec(shape, lambda i: (0,) * len(shape))


def _rows(bm, *rest):
    return pl.BlockSpec((bm,) + rest, lambda i: (i,) + (0,) * len(rest))


def _rows3(bm, d):
    return pl.BlockSpec((2, bm, d), lambda i: (0, i, 0))


def _bfpack(a):
    b = a.astype(jnp.bfloat16)
    return jax.lax.bitcast_convert_type(
        b.reshape(b.shape[:-1] + (b.shape[-1] // 2, 2)), jnp.float32)


def kernel(x, edge_index, Wp_in, bp_in, Wl_e, Wr_e, att_e, b_e,
           Wl_d, Wr_d, att_d, b_d, Wp_out, bp_out):
    f32 = jnp.float32
    bf16 = jnp.bfloat16
    src = edge_index[0]
    dst = edge_index[1]
    pad_idx = N + (jnp.arange(EP - E, dtype=jnp.int32) % (NP - N))
    srcp = jnp.concatenate([src, pad_idx])
    dstp = jnp.concatenate([dst, pad_idx])
    CE, CD = 80, 32
    ep_e = jnp.stack([srcp.reshape(-1, CE), dstp.reshape(-1, CE)], axis=1)
    ep_d = jnp.stack([srcp.reshape(-1, CD), dstp.reshape(-1, CD)], axis=1)
    x_pad = jnp.pad(x, ((0, NP - N), (0, 0)))

    m, cvec = pl.pallas_call(
        _prep_body,
        out_shape=[jax.ShapeDtypeStruct((D_IN, D1), f32),
                   jax.ShapeDtypeStruct((1, D_IN), f32)],
    )(Wp_out, Wl_d, b_d.reshape(1, D0), bp_out.reshape(1, D_IN))

    BM = 512
    grid = (NP // BM,)
    xl_e, xr_e = pl.pallas_call(
        _enc_tables_body,
        grid=grid,
        in_specs=[_rows(BM, D_IN), _full((D0, D_IN)), _full((1, D0)),
                  _full((D1, D0)), _full((D1, D0))],
        out_specs=[_rows(BM, D1), _rows(BM, D1)],
        out_shape=[jax.ShapeDtypeStruct((NP, D1), f32),
                   jax.ShapeDtypeStruct((NP, D1), f32)],
    )(x_pad, Wp_in, bp_in.reshape(1, D0), Wl_e, Wr_e)

    p_e, d_e = _make_edge_agg(D1, CE, False)(xl_e, xr_e, xl_e, ep_e, att_e)
    d_e = d_e.reshape(NC, NP, 1)

    z_pad, xl_d, xr_d, y = pl.pallas_call(
        _combine_enc_body,
        grid=grid,
        in_specs=[_rows3(BM, D1), _rows3(BM, 1), _rows(BM, D1), _rows(BM, D1),
                  _full((1, D1)), _full((1, D1)),
                  _full((D0, D1)), _full((D0, D1)), _full((D_IN, D1))],
        out_specs=[_rows(BM, D1), _rows(BM, D0), _rows(BM, D0),
                   _rows(BM, D_IN)],
        out_shape=[jax.ShapeDtypeStruct((NP, D1), f32),
                   jax.ShapeDtypeStruct((NP, D0), f32),
                   jax.ShapeDtypeStruct((NP, D0), f32),
                   jax.ShapeDtypeStruct((NP, D_IN), f32)],
    )(p_e, d_e, xl_e, xr_e, att_e.reshape(1, D1), b_e.reshape(1, D1),
      Wl_d, Wr_d, m)

    p_d, d_d = _make_edge_agg(D0, CD, True)(_bfpack(xl_d), _bfpack(xr_d),
                                            y, ep_d, _bfpack(att_d))
    d_d = d_d.reshape(NC, NP, 1)

    BF = 400
    x_hat = pl.pallas_call(
        _final_body,
        grid=(N // BF,),
        in_specs=[_rows3(BF, D_IN), _rows3(BF, 1), _rows(BF, D0),
                  _rows(BF, D0), _rows(BF, D_IN),
                  _full((1, D0)), _full((1, D_IN))],
        out_specs=_rows(BF, D_IN),
        out_shape=jax.ShapeDtypeStruct((N, D_IN), f32),
    )(p_d, d_d, xl_d, xr_d, y, att_d.reshape(1, D0), cvec)

    return (x_hat, z_pad[:N])


# speedup vs baseline: 16.6855x; 1.0250x over previous
import functools

import jax
import jax.numpy as jnp
from jax import lax
from jax.experimental import pallas as pl
from jax.experimental.pallas import tpu as pltpu
from jax.experimental.pallas import tpu_sc as plsc

N = 10000
E = 320000
D_IN = 128
D0 = 256
D1 = 128

NC, NS, L = 2, 16, 16
NW = NC * NS
NP = 10240
EP = 327680
EPW = EP // NW
RPT = NP // NS

NEG = 0.2
EPS = 1e-16


def _make_edge_agg(ds, C, packed):
    nj2 = ds // (2 * L)
    nj = ds // L
    na = 128 // L
    NCHUNK = EPW // C
    assert NCHUNK % 2 == 0 and C % L == 0

    def body(tlb, trb, ta, ep, att_h,
             num_o, den_o,
             *scratch):
        if packed:
            (p0, p1, l0, l1, r0, r1, a0, a1, att_v, g_v,
             acc_sp, den_sp, sem_g, sem_i, sem_s) = scratch
        else:
            (p0, p1, l0, l1, r0, r1, att_v, g_v,
             acc_sp, den_sp, sem_g, sem_i, sem_s) = scratch
            a0, a1 = l0, l1
        sd = (p0, p1)
        rl = (l0, l1)
        rr = (r0, r1)
        ra = (a0, a1)
        cid = lax.axis_index("c")
        sid = lax.axis_index("s")
        wid = cid * NS + sid
        rbase = wid * NCHUNK

        zv = jnp.zeros((L,), jnp.float32)

        def zr(r, _):
            for j in range(na):
                ra[0][r, pl.ds(j * L, L)] = zv
            return 0
        lax.fori_loop(0, C, zr, 0)
        def zc(k, _):
            g_v[pl.ds(k * L, L)] = zv
            return 0
        lax.fori_loop(0, C // L, zc, 0)

        for k in range(RPT // C):
            pltpu.sync_copy(ra[0], acc_sp.at[pl.ds(sid * RPT + k * C, C)])
            pltpu.sync_copy(g_v, den_sp.at[pl.ds(sid * RPT + k * C, C)])

        pltpu.sync_copy(att_h, att_v)
        if packed:
            att_c = [plsc.bitcast(att_v[pl.ds(j * L, L)], jnp.bfloat16)
                     for j in range(nj2)]
        else:
            att_c = [att_v[pl.ds(j * L, L)] for j in range(nj)]
        lane = lax.iota(jnp.int32, L)

        plsc.subcore_barrier()

        def idx_start(i, b):
            row = rbase + lax.rem(i, NCHUNK)
            pltpu.async_copy(ep.at[row], sd[b], sem_i)

        def idx_wait(b):
            pltpu.make_async_copy(ep.at[0], sd[b], sem_i).wait()

        def gather_start(b):
            pltpu.async_copy(tlb.at[sd[b].at[0]], rl[b], sem_g)
            pltpu.async_copy(trb.at[sd[b].at[1]], rr[b], sem_g)
            if packed:
                pltpu.async_copy(ta.at[sd[b].at[0]], ra[b], sem_g)

        def gather_wait(b):
            pltpu.make_async_copy(tlb.at[sd[b].at[0]], rl[b], sem_g).wait()
            pltpu.make_async_copy(trb.at[sd[b].at[1]], rr[b], sem_g).wait()
            if packed:
                pltpu.make_async_copy(ta.at[sd[b].at[0]], ra[b], sem_g).wait()

        def compute(b):
            rows_l, rows_r, agg = rl[b], rr[b], ra[b]

            def grp(cb, _):
                g_acc = jnp.zeros((L,), jnp.float32)
                for k in range(L):
                    c = cb * L + k
                    acc = None
                    if packed:
                        for j in range(nj2):
                            a = plsc.bitcast(rows_l[c, pl.ds(j * L, L)],
                                             jnp.bfloat16)
                            bb = plsc.bitcast(rows_r[c, pl.ds(j * L, L)],
                                              jnp.bfloat16)
                            t = a + bb
                            t = jnp.maximum(t, NEG * t)
                            q0, q1 = plsc.unpack(
                                t * att_c[j],
                                format=plsc.PackFormat.INTERLEAVED)
                            acc = q0 + q1 if acc is None else acc + q0 + q1
                    else:
                        for j in range(nj):
                            a = rows_l[c, pl.ds(j * L, L)]
                            bb = rows_r[c, pl.ds(j * L, L)]
                            t = a + bb
                            t = jnp.maximum(t, NEG * t)
                            acc = (t * att_c[j] if acc is None
                                   else acc + t * att_c[j])
                    gb = jnp.exp(jnp.full((L,), jnp.sum(acc)))
                    g_acc = jnp.where(lane == k, gb, g_acc)
                    for j in range(na):
                        agg[c, pl.ds(j * L, L)] = agg[c, pl.ds(j * L, L)] * gb
                g_v[pl.ds(cb * L, L)] = g_acc
                return 0
            lax.fori_loop(0, C // L, grp, 0)

        idx_start(0, 0)
        idx_wait(0)
        gather_start(0)
        idx_start(1, 1)

        def pair(g2, _):
            for b in (0, 1):
                i = 2 * g2 + b
                nb = (b + 1) % 2
                gather_wait(b)
                idx_wait(nb)
                gather_start(nb)
                compute(b)
                c1 = pltpu.async_copy(ra[b], acc_sp.at[sd[b].at[1]],
                                      sem_s, add=True)
                c2 = pltpu.async_copy(g_v, den_sp.at[sd[b].at[1]],
                                      sem_s, add=True)
                c1.wait()
                c2.wait()
                idx_start(i + 2, b)
            return 0
        lax.fori_loop(0, NCHUNK // 2, pair, 0)

        gather_wait(0)
        idx_wait(1)

        plsc.subcore_barrier()

        pltpu.sync_copy(acc_sp.at[pl.ds(sid * RPT, RPT)],
                        num_o.at[cid, pl.ds(sid * RPT, RPT)])
        pltpu.sync_copy(den_sp.at[pl.ds(sid * RPT, RPT)],
                        den_o.at[cid, pl.ds(sid * RPT, RPT)])

    mesh = plsc.VectorSubcoreMesh(core_axis_name="c", subcore_axis_name="s",
                                  num_cores=NC, num_subcores=NS)
    dsw = ds // 2 if packed else ds
    scratch = [
        pltpu.VMEM((2, C), jnp.int32),
        pltpu.VMEM((2, C), jnp.int32),
        pltpu.VMEM((C, dsw), jnp.float32),
        pltpu.VMEM((C, dsw), jnp.float32),
        pltpu.VMEM((C, dsw), jnp.float32),
        pltpu.VMEM((C, dsw), jnp.float32),
        pltpu.VMEM((C, 128), jnp.float32),
        pltpu.VMEM((C, 128), jnp.float32),
        pltpu.VMEM((dsw,), jnp.float32),
        pltpu.VMEM((C,), jnp.float32),
        pltpu.VMEM_SHARED((NP, 128), jnp.float32),
        pltpu.VMEM_SHARED((NP,), jnp.float32),
        pltpu.SemaphoreType.DMA,
        pltpu.SemaphoreType.DMA,
        pltpu.SemaphoreType.DMA,
    ]
    if not packed:
        del scratch[6:8]
    return pl.kernel(
        body,
        out_type=[jax.ShapeDtypeStruct((NC, NP, 128), jnp.float32),
                  jax.ShapeDtypeStruct((NC, NP), jnp.float32)],
        mesh=mesh,
        compiler_params=pltpu.CompilerParams(needs_layout_passes=False),
        scratch_types=scratch,
        name=f"edge_agg_d{ds}",
    )


def _dgt(a, b):
    return lax.dot_general(a, b, (((1,), (1,)), ((), ())),
                           preferred_element_type=jnp.float32)


def _enc_tables_body(x, wp_in, bp_in, wl_e, wr_e, wp_out, wl_d, b_d, bp_out,
                     xl_o, xr_o, m_o, cvec_o):
    h = _dgt(x[...], wp_in[...]) + bp_in[...]
    xl_o[...] = _dgt(h, wl_e[...])
    xr_o[...] = _dgt(h, wr_e[...])
    m_o[...] = jnp.dot(wp_out[...], wl_d[...],
                       preferred_element_type=jnp.float32)
    cvec_o[...] = _dgt(b_d[...], wp_out[...]) + bp_out[...]


def _combine_enc_body(p, den, xl, xr, att, b_e, wl_d, wr_d, m,
                      z_o, xld_o, xrd_o, y_o):
    t = xl[...] + xr[...]
    t = jnp.maximum(t, NEG * t)
    g = jnp.exp(jnp.sum(t * att[...], axis=1, keepdims=True))
    num = p[0] + p[1] + g * xl[...]
    dent = den[0] + den[1] + g + EPS
    z = num / dent + b_e[...]
    z_o[...] = z
    xld_o[...] = _dgt(z, wl_d[...])
    xrd_o[...] = _dgt(z, wr_d[...])
    y_o[...] = _dgt(z, m[...])


def _final_body(p, den, xl, xr, y, att, cvec, xhat_o):
    t = xl[...] + xr[...]
    t = jnp.maximum(t, NEG * t)
    g = jnp.exp(jnp.sum(t * att[...], axis=1, keepdims=True))
    num = p[0] + p[1] + g * y[...]
    dent = den[0] + den[1] + g + EPS
    xhat_o[...] = num / dent + cvec[...]


def _full(shape):
    return pl.BlockSpec(shape, lambda i: (0,) * len(shape))


def _rows(bm, *rest):
    return pl.BlockSpec((bm,) + rest, lambda i: (i,) + (0,) * len(rest))


def _rows3(bm, d):
    return pl.BlockSpec((2, bm, d), lambda i: (0, i, 0))


def _bfpack(a):
    b = a.astype(jnp.bfloat16)
    return jax.lax.bitcast_convert_type(
        b.reshape(b.shape[:-1] + (b.shape[-1] // 2, 2)), jnp.float32)


def kernel(x, edge_index, Wp_in, bp_in, Wl_e, Wr_e, att_e, b_e,
           Wl_d, Wr_d, att_d, b_d, Wp_out, bp_out):
    f32 = jnp.float32
    bf16 = jnp.bfloat16
    src = edge_index[0]
    dst = edge_index[1]
    pad_idx = N + (jnp.arange(EP - E, dtype=jnp.int32) % (NP - N))
    srcp = jnp.concatenate([src, pad_idx])
    dstp = jnp.concatenate([dst, pad_idx])
    CE, CD = 80, 32
    ep_e = jnp.stack([srcp.reshape(-1, CE), dstp.reshape(-1, CE)], axis=1)
    ep_d = jnp.stack([srcp.reshape(-1, CD), dstp.reshape(-1, CD)], axis=1)
    x_pad = jnp.pad(x, ((0, NP - N), (0, 0)))

    BM = 512
    grid = (NP // BM,)
    xl_e, xr_e, m, cvec = pl.pallas_call(
        _enc_tables_body,
        grid=grid,
        in_specs=[_rows(BM, D_IN), _full((D0, D_IN)), _full((1, D0)),
                  _full((D1, D0)), _full((D1, D0)),
                  _full((D_IN, D0)), _full((D0, D1)), _full((1, D0)),
                  _full((1, D_IN))],
        out_specs=[_rows(BM, D1), _rows(BM, D1),
                   _full((D_IN, D1)), _full((1, D_IN))],
        out_shape=[jax.ShapeDtypeStruct((NP, D1), f32),
                   jax.ShapeDtypeStruct((NP, D1), f32),
                   jax.ShapeDtypeStruct((D_IN, D1), f32),
                   jax.ShapeDtypeStruct((1, D_IN), f32)],
    )(x_pad, Wp_in, bp_in.reshape(1, D0), Wl_e, Wr_e,
      Wp_out, Wl_d, b_d.reshape(1, D0), bp_out.reshape(1, D_IN))

    p_e, d_e = _make_edge_agg(D1, CE, False)(xl_e, xr_e, xl_e, ep_e, att_e)
    d_e = d_e.reshape(NC, NP, 1)

    z_pad, xl_d, xr_d, y = pl.pallas_call(
        _combine_enc_body,
        grid=grid,
        in_specs=[_rows3(BM, D1), _rows3(BM, 1), _rows(BM, D1), _rows(BM, D1),
                  _full((1, D1)), _full((1, D1)),
                  _full((D0, D1)), _full((D0, D1)), _full((D_IN, D1))],
        out_specs=[_rows(BM, D1), _rows(BM, D0), _rows(BM, D0),
                   _rows(BM, D_IN)],
        out_shape=[jax.ShapeDtypeStruct((NP, D1), f32),
                   jax.ShapeDtypeStruct((NP, D0), f32),
                   jax.ShapeDtypeStruct((NP, D0), f32),
                   jax.ShapeDtypeStruct((NP, D_IN), f32)],
    )(p_e, d_e, xl_e, xr_e, att_e.reshape(1, D1), b_e.reshape(1, D1),
      Wl_d, Wr_d, m)

    p_d, d_d = _make_edge_agg(D0, CD, True)(_bfpack(xl_d), _bfpack(xr_d),
                                            y, ep_d, _bfpack(att_d))
    d_d = d_d.reshape(NC, NP, 1)

    BF = 400
    x_hat = pl.pallas_call(
        _final_body,
        grid=(N // BF,),
        in_specs=[_rows3(BF, D_IN), _rows3(BF, 1), _rows(BF, D0),
                  _rows(BF, D0), _rows(BF, D_IN),
                  _full((1, D0)), _full((1, D_IN))],
        out_specs=_rows(BF, D_IN),
        out_shape=jax.ShapeDtypeStruct((N, D_IN), f32),
    )(p_d, d_d, xl_d, xr_d, y, att_d.reshape(1, D0), cvec)

    return (x_hat, z_pad[:N])


# speedup vs baseline: 16.6890x; 1.0002x over previous
import jax
import jax.numpy as jnp
from jax import lax
from jax.experimental import pallas as pl
from jax.experimental.pallas import tpu as pltpu
from jax.experimental.pallas import tpu_sc as plsc

N = 10000
E = 320000
D_IN = 128
D0 = 256
D1 = 128

NC, NS, L = 2, 16, 16
NW = NC * NS
NP = 10240
EP = 327680
EPW = EP // NW
RPT = NP // NS

NEG = 0.2
EPS = 1e-16


def _make_edge_agg(ds, C, packed):
    nj2 = ds // (2 * L)
    nj = ds // L
    na = 128 // L
    NCHUNK = EPW // C
    assert NCHUNK % 2 == 0 and C % L == 0

    def body(tlb, trb, ta, ep, att_h,
             num_o, den_o,
             *scratch):
        if packed:
            (p0, p1, l0, l1, r0, r1, a0, a1, att_v, g_v,
             acc_sp, den_sp, sem_g, sem_i, sem_s) = scratch
        else:
            (p0, p1, l0, l1, r0, r1, att_v, g_v,
             acc_sp, den_sp, sem_g, sem_i, sem_s) = scratch
            a0, a1 = l0, l1
        sd = (p0, p1)
        rl = (l0, l1)
        rr = (r0, r1)
        ra = (a0, a1)
        cid = lax.axis_index("c")
        sid = lax.axis_index("s")
        wid = cid * NS + sid
        rbase = wid * NCHUNK

        zv = jnp.zeros((L,), jnp.float32)

        def zr(r, _):
            for j in range(na):
                ra[0][r, pl.ds(j * L, L)] = zv
            return 0
        lax.fori_loop(0, C, zr, 0)
        def zc(k, _):
            g_v[pl.ds(k * L, L)] = zv
            return 0
        lax.fori_loop(0, C // L, zc, 0)

        for k in range(RPT // C):
            pltpu.sync_copy(ra[0], acc_sp.at[pl.ds(sid * RPT + k * C, C)])
            pltpu.sync_copy(g_v, den_sp.at[pl.ds(sid * RPT + k * C, C)])

        pltpu.sync_copy(att_h, att_v)
        if packed:
            att_c = [plsc.bitcast(att_v[pl.ds(j * L, L)], jnp.bfloat16)
                     for j in range(nj2)]
        else:
            att_c = [att_v[pl.ds(j * L, L)] for j in range(nj)]
        lane = lax.iota(jnp.int32, L)

        plsc.subcore_barrier()

        def idx_start(i, b):
            row = rbase + lax.rem(i, NCHUNK)
            pltpu.async_copy(ep.at[row], sd[b], sem_i)

        def idx_wait(b):
            pltpu.make_async_copy(ep.at[0], sd[b], sem_i).wait()

        def gather_start(b):
            pltpu.async_copy(tlb.at[sd[b].at[0]], rl[b], sem_g)
            pltpu.async_copy(trb.at[sd[b].at[1]], rr[b], sem_g)
            if packed:
                pltpu.async_copy(ta.at[sd[b].at[0]], ra[b], sem_g)

        def gather_wait(b):
            pltpu.make_async_copy(tlb.at[sd[b].at[0]], rl[b], sem_g).wait()
            pltpu.make_async_copy(trb.at[sd[b].at[1]], rr[b], sem_g).wait()
            if packed:
                pltpu.make_async_copy(ta.at[sd[b].at[0]], ra[b], sem_g).wait()

        def compute(b):
            rows_l, rows_r, agg = rl[b], rr[b], ra[b]

            def grp(cb, _):
                g_acc = jnp.zeros((L,), jnp.float32)
                for k in range(L):
                    c = cb * L + k
                    acc = None
                    if packed:
                        for j in range(nj2):
                            a = plsc.bitcast(rows_l[c, pl.ds(j * L, L)],
                                             jnp.bfloat16)
                            bb = plsc.bitcast(rows_r[c, pl.ds(j * L, L)],
                                              jnp.bfloat16)
                            t = a + bb
                            t = jnp.maximum(t, NEG * t)
                            q0, q1 = plsc.unpack(
                                t * att_c[j],
                                format=plsc.PackFormat.INTERLEAVED)
                            acc = q0 + q1 if acc is None else acc + q0 + q1
                    else:
                        for j in range(nj):
                            a = rows_l[c, pl.ds(j * L, L)]
                            bb = rows_r[c, pl.ds(j * L, L)]
                            t = a + bb
                            t = jnp.maximum(t, NEG * t)
                            acc = (t * att_c[j] if acc is None
                                   else acc + t * att_c[j])
                    gb = jnp.exp(jnp.full((L,), jnp.sum(acc)))
                    g_acc = jnp.where(lane == k, gb, g_acc)
                    for j in range(na):
                        agg[c, pl.ds(j * L, L)] = agg[c, pl.ds(j * L, L)] * gb
                g_v[pl.ds(cb * L, L)] = g_acc
                return 0
            lax.fori_loop(0, C // L, grp, 0)

        idx_start(0, 0)
        idx_wait(0)
        gather_start(0)
        idx_start(1, 1)

        def pair(g2, _):
            for b in (0, 1):
                i = 2 * g2 + b
                nb = (b + 1) % 2
                gather_wait(b)
                idx_wait(nb)
                gather_start(nb)
                compute(b)
                c1 = pltpu.async_copy(ra[b], acc_sp.at[sd[b].at[1]],
                                      sem_s, add=True)
                c2 = pltpu.async_copy(g_v, den_sp.at[sd[b].at[1]],
                                      sem_s, add=True)
                c1.wait()
                c2.wait()
                idx_start(i + 2, b)
            return 0
        lax.fori_loop(0, NCHUNK // 2, pair, 0)

        gather_wait(0)
        idx_wait(1)

        plsc.subcore_barrier()

        pltpu.sync_copy(acc_sp.at[pl.ds(sid * RPT, RPT)],
                        num_o.at[cid, pl.ds(sid * RPT, RPT)])
        pltpu.sync_copy(den_sp.at[pl.ds(sid * RPT, RPT)],
                        den_o.at[cid, pl.ds(sid * RPT, RPT)])

    mesh = plsc.VectorSubcoreMesh(core_axis_name="c", subcore_axis_name="s",
                                  num_cores=NC, num_subcores=NS)
    dsw = ds // 2 if packed else ds
    scratch = [
        pltpu.VMEM((2, C), jnp.int32),
        pltpu.VMEM((2, C), jnp.int32),
        pltpu.VMEM((C, dsw), jnp.float32),
        pltpu.VMEM((C, dsw), jnp.float32),
        pltpu.VMEM((C, dsw), jnp.float32),
        pltpu.VMEM((C, dsw), jnp.float32),
        pltpu.VMEM((C, 128), jnp.float32),
        pltpu.VMEM((C, 128), jnp.float32),
        pltpu.VMEM((dsw,), jnp.float32),
        pltpu.VMEM((C,), jnp.float32),
        pltpu.VMEM_SHARED((NP, 128), jnp.float32),
        pltpu.VMEM_SHARED((NP,), jnp.float32),
        pltpu.SemaphoreType.DMA,
        pltpu.SemaphoreType.DMA,
        pltpu.SemaphoreType.DMA,
    ]
    if not packed:
        del scratch[6:8]
    return pl.kernel(
        body,
        out_type=[jax.ShapeDtypeStruct((NC, NP, 128), jnp.float32),
                  jax.ShapeDtypeStruct((NC, NP), jnp.float32)],
        mesh=mesh,
        compiler_params=pltpu.CompilerParams(needs_layout_passes=False),
        scratch_types=scratch,
        name=f"edge_agg_d{ds}",
    )


def _dgt(a, b):
    return lax.dot_general(a, b, (((1,), (1,)), ((), ())),
                           preferred_element_type=jnp.float32)


def _enc_tables_body(x, wp_in, bp_in, wl_e, wr_e, wp_out, wl_d, b_d, bp_out,
                     xl_o, xr_o, m_o, cvec_o):
    h = _dgt(x[...], wp_in[...]) + bp_in[...]
    xl_o[...] = _dgt(h, wl_e[...])
    xr_o[...] = _dgt(h, wr_e[...])
    m_o[...] = jnp.dot(wp_out[...], wl_d[...],
                       preferred_element_type=jnp.float32)
    cvec_o[...] = _dgt(b_d[...], wp_out[...]) + bp_out[...]


def _combine_enc_body(p, den, xl, xr, att, b_e, wl_d, wr_d, m,
                      z_o, xld_o, xrd_o, y_o):
    t = xl[...] + xr[...]
    t = jnp.maximum(t, NEG * t)
    g = jnp.exp(jnp.sum(t * att[...], axis=1, keepdims=True))
    num = p[0] + p[1] + g * xl[...]
    dent = den[0] + den[1] + g + EPS
    z = num / dent + b_e[...]
    z_o[...] = z
    xld_o[...] = _dgt(z, wl_d[...])
    xrd_o[...] = _dgt(z, wr_d[...])
    y_o[...] = _dgt(z, m[...])


def _final_body(p, den, xl, xr, y, att, cvec, xhat_o):
    t = xl[...] + xr[...]
    t = jnp.maximum(t, NEG * t)
    g = jnp.exp(jnp.sum(t * att[...], axis=1, keepdims=True))
    num = p[0] + p[1] + g * y[...]
    dent = den[0] + den[1] + g + EPS
    xhat_o[...] = num / dent + cvec[...]


def _full(shape):
    return pl.BlockSpec(shape, lambda i: (0,) * len(shape))


def _rows(bm, *rest):
    return pl.BlockSpec((bm,) + rest, lambda i: (i,) + (0,) * len(rest))


def _rows3(bm, d):
    return pl.BlockSpec((2, bm, d), lambda i: (0, i, 0))


def _bfpack(a):
    b = a.astype(jnp.bfloat16)
    return jax.lax.bitcast_convert_type(
        b.reshape(b.shape[:-1] + (b.shape[-1] // 2, 2)), jnp.float32)


def kernel(x, edge_index, Wp_in, bp_in, Wl_e, Wr_e, att_e, b_e,
           Wl_d, Wr_d, att_d, b_d, Wp_out, bp_out):
    f32 = jnp.float32
    src = edge_index[0]
    dst = edge_index[1]
    pad_idx = N + (jnp.arange(EP - E, dtype=jnp.int32) % (NP - N))
    srcp = jnp.concatenate([src, pad_idx])
    dstp = jnp.concatenate([dst, pad_idx])
    CE, CD = 80, 32
    ep_e = jnp.stack([srcp.reshape(-1, CE), dstp.reshape(-1, CE)], axis=1)
    ep_d = jnp.stack([srcp.reshape(-1, CD), dstp.reshape(-1, CD)], axis=1)
    x_pad = jnp.pad(x, ((0, NP - N), (0, 0)))

    BM = 512
    grid = (NP // BM,)
    xl_e, xr_e, m, cvec = pl.pallas_call(
        _enc_tables_body,
        grid=grid,
        in_specs=[_rows(BM, D_IN), _full((D0, D_IN)), _full((1, D0)),
                  _full((D1, D0)), _full((D1, D0)),
                  _full((D_IN, D0)), _full((D0, D1)), _full((1, D0)),
                  _full((1, D_IN))],
        out_specs=[_rows(BM, D1), _rows(BM, D1),
                   _full((D_IN, D1)), _full((1, D_IN))],
        out_shape=[jax.ShapeDtypeStruct((NP, D1), f32),
                   jax.ShapeDtypeStruct((NP, D1), f32),
                   jax.ShapeDtypeStruct((D_IN, D1), f32),
                   jax.ShapeDtypeStruct((1, D_IN), f32)],
    )(x_pad, Wp_in, bp_in.reshape(1, D0), Wl_e, Wr_e,
      Wp_out, Wl_d, b_d.reshape(1, D0), bp_out.reshape(1, D_IN))

    p_e, d_e = _make_edge_agg(D1, CE, False)(xl_e, xr_e, xl_e, ep_e, att_e)
    d_e = d_e.reshape(NC, NP, 1)

    z_pad, xl_d, xr_d, y = pl.pallas_call(
        _combine_enc_body,
        grid=grid,
        in_specs=[_rows3(BM, D1), _rows3(BM, 1), _rows(BM, D1), _rows(BM, D1),
                  _full((1, D1)), _full((1, D1)),
                  _full((D0, D1)), _full((D0, D1)), _full((D_IN, D1))],
        out_specs=[_rows(BM, D1), _rows(BM, D0), _rows(BM, D0),
                   _rows(BM, D_IN)],
        out_shape=[jax.ShapeDtypeStruct((NP, D1), f32),
                   jax.ShapeDtypeStruct((NP, D0), f32),
                   jax.ShapeDtypeStruct((NP, D0), f32),
                   jax.ShapeDtypeStruct((NP, D_IN), f32)],
    )(p_e, d_e, xl_e, xr_e, att_e.reshape(1, D1), b_e.reshape(1, D1),
      Wl_d, Wr_d, m)

    p_d, d_d = _make_edge_agg(D0, CD, True)(_bfpack(xl_d), _bfpack(xr_d),
                                            y, ep_d, _bfpack(att_d))
    d_d = d_d.reshape(NC, NP, 1)

    BF = 400
    x_hat = pl.pallas_call(
        _final_body,
        grid=(N // BF,),
        in_specs=[_rows3(BF, D_IN), _rows3(BF, 1), _rows(BF, D0),
                  _rows(BF, D0), _rows(BF, D_IN),
                  _full((1, D0)), _full((1, D_IN))],
        out_specs=_rows(BF, D_IN),
        out_shape=jax.ShapeDtypeStruct((N, D_IN), f32),
    )(p_d, d_d, xl_d, xr_d, y, att_d.reshape(1, D0), cvec)

    return (x_hat, z_pad[:N])
